# baseline (speedup 1.0000x reference)
import functools

import jax
import jax.numpy as jnp
from jax import lax
from jax.experimental import pallas as pl
from jax.experimental.pallas import tpu as pltpu
from jax.experimental.pallas import tpu_sc as plsc

_NUM_BINS = 15
_NUM_CLASSES = 19
_LANES = 16
_NC = 2
_NS = 16
_NW = _NC * _NS
_HR = 2
_H_SC = 192
_TB = 8


def _tree_sum(xs):
    xs = list(xs)
    while len(xs) > 1:
        nxt = [a + b for a, b in zip(xs[0::2], xs[1::2])]
        if len(xs) % 2:
            nxt.append(xs[-1])
        xs = nxt
    return xs[0]


def _sc_body(logits_hbm, vf_hbm, out_hbm, in_a, in_b, out_a, out_b, vf_v,
             isem_a, isem_b, osem_a, osem_b):
    C = _NUM_CLASSES
    B = logits_hbm.shape[0]
    W = logits_hbm.shape[3]
    wid = lax.axis_index("s") * _NC + lax.axis_index("c")
    pltpu.sync_copy(vf_hbm, vf_v)

    stripes = _H_SC // _HR
    slabs = (B * stripes) // _NW

    def src_at(t):
        g = wid * slabs + t
        b = g // stripes
        h0 = (g % stripes) * _HR
        return logits_hbm.at[b, :, pl.ds(h0, _HR), :]

    def dst_at(t):
        g = wid * slabs + t
        b = g // stripes
        h0 = (g % stripes) * _HR
        return out_hbm.at[b, :, pl.ds(h0, _HR), :]

    def process(in_v, out_v, hh, off):
        es = [jnp.exp(in_v[c, hh, pl.ds(off, _LANES)]) for c in range(C)]
        r = jnp.float32(_NUM_BINS) / _tree_sum(es)
        cal = []
        for c in range(C):
            bidx = (es[c] * r).astype(jnp.int32)
            cal.append(plsc.load_gather(vf_v, [bidx + c * 16]))
        t = _tree_sum(cal)
        t = jnp.where(t == 0.0, jnp.float32(1.0), t)
        it = jnp.float32(1.0) / t
        for c in range(C):
            out_v[c, hh, pl.ds(off, _LANES)] = cal[c] * it

    bufs = ((in_a, out_a, isem_a, osem_a), (in_b, out_b, isem_b, osem_b))

    for p in range(2):
        in_v, _, isem, _ = bufs[p]
        pltpu.async_copy(src_at(p), in_v, isem)

    def pair_body(k, carry):
        for p in range(2):
            in_v, out_v, isem, osem = bufs[p]
            t = k * 2 + p
            pltpu.make_async_copy(src_at(t), in_v, isem).wait()

            @pl.when(t >= 2)
            def _():
                pltpu.make_async_copy(out_v, dst_at(t - 2), osem).wait()

            def h_loop(hh, c2):
                def v_loop(vv, c3):
                    process(in_v, out_v, hh, vv * _LANES)
                    return c3
                lax.fori_loop(0, W // _LANES, v_loop, c2)
                return c2

            lax.fori_loop(0, _HR, h_loop, 0)
            pltpu.async_copy(out_v, dst_at(t), osem)

            @pl.when(t + 2 < slabs)
            def _():
                pltpu.async_copy(src_at(t + 2), in_v, isem)
        return carry

    lax.fori_loop(0, slabs // 2, pair_body, 0)

    for p in range(2):
        _, out_v, _, osem = bufs[p]
        pltpu.make_async_copy(out_v, dst_at(slabs - 2 + p), osem).wait()


def _tc_body(x_ref, vf_ref, o_ref):
    C = _NUM_CLASSES
    x = x_ref[0]
    e = jnp.exp(x)
    s = jnp.sum(e, axis=0)
    r = jnp.float32(_NUM_BINS) / s
    cals = []
    t = None
    for c in range(C):
        bc = (e[c] * r).astype(jnp.int32)
        a = jnp.full(bc.shape, vf_ref[c, 0], dtype=jnp.float32)
        for i in range(1, 16):
            a = jnp.where(bc >= i, vf_ref[c, i], a)
        cals.append(a)
        t = a if t is None else t + a
    t = jnp.where(t == 0.0, jnp.float32(1.0), t)
    it = jnp.float32(1.0) / t
    for c in range(C):
        o_ref[0, c] = cals[c] * it


def kernel(logits, val_freqs):
    B, C, H, W = logits.shape
    vf = jnp.concatenate([val_freqs, val_freqs[:, -1:]], axis=1).reshape(-1)

    mesh = plsc.VectorSubcoreMesh(core_axis_name="c", subcore_axis_name="s")
    sc_call = functools.partial(
        pl.kernel,
        out_type=jax.ShapeDtypeStruct((B, C, _H_SC, W), jnp.float32),
        mesh=mesh,
        scratch_types=[
            pltpu.VMEM((C, _HR, W), jnp.float32),
            pltpu.VMEM((C, _HR, W), jnp.float32),
            pltpu.VMEM((C, _HR, W), jnp.float32),
            pltpu.VMEM((C, _HR, W), jnp.float32),
            pltpu.VMEM((C * 16,), jnp.float32),
            pltpu.SemaphoreType.DMA,
            pltpu.SemaphoreType.DMA,
            pltpu.SemaphoreType.DMA,
            pltpu.SemaphoreType.DMA,
        ],
        compiler_params=pltpu.CompilerParams(needs_layout_passes=False),
    )(_sc_body)
    sc_out = sc_call(logits, vf)

    h_tc = H - _H_SC
    tc_out = pl.pallas_call(
        _tc_body,
        out_shape=jax.ShapeDtypeStruct((B, C, h_tc, W), jnp.float32),
        grid=(B, h_tc // _TB),
        in_specs=[
            pl.BlockSpec((1, C, _TB, W),
                         lambda b, h: (b, 0, h + _H_SC // _TB, 0)),
            pl.BlockSpec(memory_space=pltpu.SMEM),
        ],
        out_specs=pl.BlockSpec((1, C, _TB, W), lambda b, h: (b, 0, h, 0)),
    )(logits, vf.reshape(C, 16))
    return jnp.concatenate([sc_out, tc_out], axis=2)


# speedup vs baseline: 1.1092x; 1.1092x over previous
import functools

import jax
import jax.numpy as jnp
from jax import lax
from jax.experimental import pallas as pl
from jax.experimental.pallas import tpu as pltpu
from jax.experimental.pallas import tpu_sc as plsc

_NUM_BINS = 15
_NUM_CLASSES = 19
_LANES = 16
_NC = 2
_NS = 16
_NW = _NC * _NS
_HR = 2
_REP = 305


def _tree_sum(xs):
    xs = list(xs)
    while len(xs) > 1:
        nxt = [a + b for a, b in zip(xs[0::2], xs[1::2])]
        if len(xs) % 2:
            nxt.append(xs[-1])
        xs = nxt
    return xs[0]


def _body(logits_hbm, vf_hbm, out_hbm, in_a, in_b, out_a, out_b, vf_v,
          isem_a, isem_b, osem_a, osem_b):
    C = _NUM_CLASSES
    B, _, H, W = logits_hbm.shape
    wid = lax.axis_index("s") * _NC + lax.axis_index("c")
    pltpu.sync_copy(vf_hbm, vf_v)

    stripes = H // _HR
    slabs = (B * stripes) // _NW
    lane_base = lax.iota(jnp.int32, _LANES) * _REP

    def src_at(t):
        g = wid * slabs + t
        b = g // stripes
        h0 = (g % stripes) * _HR
        return logits_hbm.at[b, :, pl.ds(h0, _HR), :]

    def dst_at(t):
        g = wid * slabs + t
        b = g // stripes
        h0 = (g % stripes) * _HR
        return out_hbm.at[b, :, pl.ds(h0, _HR), :]

    def process(in_v, out_v, hh, off):
        es = [jnp.exp(in_v[c, hh, pl.ds(off, _LANES)]) for c in range(C)]
        r = jnp.float32(_NUM_BINS) / _tree_sum(es)
        cal = []
        for c in range(C):
            bidx = (es[c] * r).astype(jnp.int32) + c * 16
            cal.append(plsc.load_gather(vf_v, [bidx + lane_base]))
        t = _tree_sum(cal)
        t = jnp.where(t == 0.0, jnp.float32(1.0), t)
        it = jnp.float32(1.0) / t
        for c in range(C):
            out_v[c, hh, pl.ds(off, _LANES)] = cal[c] * it

    bufs = ((in_a, out_a, isem_a, osem_a), (in_b, out_b, isem_b, osem_b))

    for p in range(2):
        in_v, _, isem, _ = bufs[p]
        pltpu.async_copy(src_at(p), in_v, isem)

    def pair_body(k, carry):
        for p in range(2):
            in_v, out_v, isem, osem = bufs[p]
            t = k * 2 + p
            pltpu.make_async_copy(src_at(t), in_v, isem).wait()

            @pl.when(t >= 2)
            def _():
                pltpu.make_async_copy(out_v, dst_at(t - 2), osem).wait()

            def h_loop(hh, c2):
                def v_loop(vv, c3):
                    process(in_v, out_v, hh, vv * _LANES)
                    return c3
                lax.fori_loop(0, W // _LANES, v_loop, c2)
                return c2

            lax.fori_loop(0, _HR, h_loop, 0)
            pltpu.async_copy(out_v, dst_at(t), osem)

            @pl.when(t + 2 < slabs)
            def _():
                pltpu.async_copy(src_at(t + 2), in_v, isem)
        return carry

    lax.fori_loop(0, slabs // 2, pair_body, 0)

    for p in range(2):
        _, out_v, _, osem = bufs[p]
        pltpu.make_async_copy(out_v, dst_at(slabs - 2 + p), osem).wait()


def kernel(logits, val_freqs):
    B, C, H, W = logits.shape
    vf = jnp.concatenate([val_freqs, val_freqs[:, -1:]], axis=1).reshape(-1)
    vf = jnp.tile(jnp.pad(vf, (0, _REP - vf.shape[0])), _LANES)

    mesh = plsc.VectorSubcoreMesh(core_axis_name="c", subcore_axis_name="s")
    call = functools.partial(
        pl.kernel,
        out_type=jax.ShapeDtypeStruct((B, C, H, W), jnp.float32),
        mesh=mesh,
        scratch_types=[
            pltpu.VMEM((C, _HR, W), jnp.float32),
            pltpu.VMEM((C, _HR, W), jnp.float32),
            pltpu.VMEM((C, _HR, W), jnp.float32),
            pltpu.VMEM((C, _HR, W), jnp.float32),
            pltpu.VMEM((_REP * _LANES,), jnp.float32),
            pltpu.SemaphoreType.DMA,
            pltpu.SemaphoreType.DMA,
            pltpu.SemaphoreType.DMA,
            pltpu.SemaphoreType.DMA,
        ],
        compiler_params=pltpu.CompilerParams(needs_layout_passes=False),
    )(_body)
    return call(logits, vf)
